# baseline (speedup 1.0000x reference)
import functools

import jax
import jax.numpy as jnp
from jax import lax
from jax.experimental import pallas as pl
from jax.experimental.pallas import tpu as pltpu
from jax.experimental.pallas import tpu_sc as plsc

_NC = 2
_NS = 16
_NW = _NC * _NS
_CH = 112
_SEC = 24
_RING = 3
_TRASH = 112


def _pad_sizes(n_edges, n_nodes):
    nch = -(-n_edges // (_NW * _CH))
    nch = -(-nch // _SEC) * _SEC
    e_pad = _NW * _CH * nch
    n_acc = ((n_nodes + _TRASH + 127) // 128) * 128
    n_deg = ((n_nodes + _TRASH + _NS * 16 - 1) // (_NS * 16)) * (_NS * 16)
    return nch, e_pad, n_acc, n_deg


def _make_deg_kernel(nch, n_acc):
    mesh = plsc.VectorSubcoreMesh(core_axis_name="c", subcore_axis_name="s")
    grp = n_acc // _NS

    @functools.partial(
        pl.kernel,
        out_type=jax.ShapeDtypeStruct((_NC, n_acc), jnp.float32),
        mesh=mesh,
        scratch_types=[
            pltpu.VMEM((nch, _CH), jnp.int32),
            pltpu.VMEM((_CH,), jnp.float32),
            pltpu.VMEM((grp,), jnp.float32),
            pltpu.VMEM_SHARED((n_acc,), jnp.float32),
        ],
    )
    def deg_kernel(dst_hbm, out_hbm, idxd, ones, zbuf, acc):
        cid = lax.axis_index("c")
        sid = lax.axis_index("s")
        wid = cid * _NS + sid
        pltpu.sync_copy(dst_hbm.at[pl.ds(wid * nch, nch)], idxd)

        def fill(i, carry):
            ones[pl.ds(i * 16, 16)] = jnp.ones((16,), jnp.float32)
            return carry
        lax.fori_loop(0, _CH // 16, fill, 0)

        def zfill(i, carry):
            zbuf[pl.ds(i * 16, 16)] = jnp.zeros((16,), jnp.float32)
            return carry
        lax.fori_loop(0, grp // 16, zfill, 0)
        pltpu.sync_copy(zbuf, acc.at[pl.ds(sid * grp, grp)])
        plsc.subcore_barrier()

        def chunk(j, carry):
            pltpu.sync_copy(ones, acc.at[idxd.at[j]], add=True)
            return carry
        lax.fori_loop(0, nch, chunk, 0)
        plsc.subcore_barrier()
        pltpu.sync_copy(acc.at[pl.ds(sid * grp, grp)],
                        out_hbm.at[cid, pl.ds(sid * grp, grp)])

    return deg_kernel


def _make_scatter_kernel(n_nodes, hid, nch, n_acc):
    mesh = plsc.VectorSubcoreMesh(core_axis_name="c", subcore_axis_name="s")
    grp = n_acc // _NS
    assert nch % _SEC == 0 and _SEC % _RING == 0 and _SEC % 8 == 0
    nsec = nch // _SEC
    assert grp % 8 == 0

    @functools.partial(
        pl.kernel,
        out_type=jax.ShapeDtypeStruct((_NC, n_acc, hid), jnp.float32),
        mesh=mesh,
        scratch_types=[
            pltpu.VMEM((_SEC, _CH), jnp.int32),
            pltpu.VMEM((_SEC, _CH), jnp.int32),
        ] + [pltpu.VMEM((_CH, hid), jnp.float32) for _ in range(_RING)]
          + [pltpu.SemaphoreType.DMA for _ in range(_RING)]
          + [pltpu.VMEM_SHARED((n_acc, hid), jnp.float32)],
    )
    def scat_kernel(g_hbm, src_hbm, dst_hbm, out_hbm,
                    idxs, idxd, *bufs_sems_acc):
        rows = bufs_sems_acc[:_RING]
        sems = bufs_sems_acc[_RING:2 * _RING]
        acc = bufs_sems_acc[2 * _RING]
        cid = lax.axis_index("c")
        sid = lax.axis_index("s")
        wid = cid * _NS + sid

        def zfill(i, carry):
            for k in range(hid // 16):
                rows[0][i, pl.ds(k * 16, 16)] = jnp.zeros((16,), jnp.float32)
            return carry
        lax.fori_loop(0, _CH, zfill, 0)
        nfull, rem = divmod(grp, _CH)
        for z in range(nfull):
            pltpu.sync_copy(
                rows[0], acc.at[pl.ds(sid * grp + z * _CH, _CH)])
        if rem:
            pltpu.sync_copy(
                rows[0].at[pl.ds(0, rem)],
                acc.at[pl.ds(sid * grp + nfull * _CH, rem)])
        plsc.subcore_barrier()

        for sec in range(nsec):
            base = wid * nch + sec * _SEC
            pltpu.sync_copy(src_hbm.at[pl.ds(base, _SEC)], idxs)
            pltpu.sync_copy(dst_hbm.at[pl.ds(base, _SEC)], idxd)
            for b in range(_RING):
                pltpu.async_copy(acc.at[idxs.at[b]], rows[b], sems[b])

            def rnd(p, carry):
                j0 = p * _RING
                for b in range(_RING):
                    j = j0 + b
                    pltpu.make_async_copy(
                        acc.at[idxs.at[j]], rows[b], sems[b]).wait()
                    pltpu.sync_copy(rows[b], acc.at[idxd.at[j]], add=True)

                    @pl.when(j + _RING < _SEC)
                    def _():
                        pltpu.async_copy(
                            acc.at[idxs.at[j + _RING]], rows[b], sems[b])
                return carry
            lax.fori_loop(0, _SEC // _RING, rnd, 0)
        plsc.subcore_barrier()
        pltpu.sync_copy(acc.at[pl.ds(sid * grp, grp)],
                        out_hbm.at[cid, pl.ds(sid * grp, grp)])

    return scat_kernel


def _tc_first(x_ref, wp_ref, bp_ref, w1_ref, degt_ref, g32_ref, *, n_nodes):
    deg = degt_ref[pl.ds(0, n_nodes), 0:1] + degt_ref[pl.ds(0, n_nodes), 1:2] + 1.0
    dinv = lax.rsqrt(deg)
    h0 = jnp.maximum(x_ref[...] @ wp_ref[...] + bp_ref[...][None, :], 0.0)
    g32_ref[...] = dinv * (h0 @ w1_ref[...])


def _tc_mid(s_ref, g_ref, b_ref, w_ref, degt_ref, g32_ref, *, n_nodes):
    deg = degt_ref[pl.ds(0, n_nodes), 0:1] + degt_ref[pl.ds(0, n_nodes), 1:2] + 1.0
    dinv = lax.rsqrt(deg)
    s = s_ref[0, pl.ds(0, n_nodes), :] + s_ref[1, pl.ds(0, n_nodes), :]
    h = jnp.maximum(dinv * (s + g_ref[...]) + b_ref[...][None, :], 0.0)
    g32_ref[...] = dinv * (h @ w_ref[...])


def _tc_last(s_ref, g_ref, b_ref, degt_ref, wc1_ref, bc1_ref, wc2_ref,
             bc2_ref, out_ref, *, n_nodes):
    deg = degt_ref[pl.ds(0, n_nodes), 0:1] + degt_ref[pl.ds(0, n_nodes), 1:2] + 1.0
    dinv = lax.rsqrt(deg)
    s = s_ref[0, pl.ds(0, n_nodes), :] + s_ref[1, pl.ds(0, n_nodes), :]
    h = jnp.maximum(dinv * (s + g_ref[...]) + b_ref[...][None, :], 0.0)
    m = jnp.sum(h, axis=0, keepdims=True) * (1.0 / n_nodes)
    hidden = jnp.maximum(m @ wc1_ref[...] + bc1_ref[...][None, :], 0.0)
    out_ref[...] = hidden @ wc2_ref[...] + bc2_ref[...][None, :]


def kernel(x, edge_index, Wp, bp, W1, b1, W2, b2, W3, b3, Wc1, bc1, Wc2, bc2):
    n = x.shape[0]
    e = edge_index.shape[1]
    hid = W1.shape[0]
    nch, e_pad, n_acc, n_deg = _pad_sizes(e, n)

    src = edge_index[0].astype(jnp.int32)
    dst = edge_index[1].astype(jnp.int32)
    pad = e_pad - e
    src_p = jnp.concatenate(
        [src, (jnp.arange(pad, dtype=jnp.int32) * 37) % n])
    dst_p = jnp.concatenate(
        [dst, n + (jnp.arange(pad, dtype=jnp.int32) % _TRASH)])
    src2d = src_p.reshape(_NW * nch, _CH)
    dst2d = dst_p.reshape(_NW * nch, _CH)

    deg_kernel = _make_deg_kernel(nch, n_deg)
    scat_kernel = _make_scatter_kernel(n, hid, nch, n_acc)

    deg_p = deg_kernel(dst2d)
    deg_t = deg_p.T

    g_shape = jax.ShapeDtypeStruct((n, hid), jnp.float32)
    g0 = pl.pallas_call(
        functools.partial(_tc_first, n_nodes=n),
        out_shape=g_shape,
    )(x, Wp, bp, W1, deg_t)

    s0 = scat_kernel(g0, src2d, dst2d)
    g1 = pl.pallas_call(
        functools.partial(_tc_mid, n_nodes=n),
        out_shape=g_shape,
    )(s0, g0, b1, W2, deg_t)

    s1 = scat_kernel(g1, src2d, dst2d)
    g2 = pl.pallas_call(
        functools.partial(_tc_mid, n_nodes=n),
        out_shape=g_shape,
    )(s1, g1, b2, W3, deg_t)

    s2 = scat_kernel(g2, src2d, dst2d)
    logits = pl.pallas_call(
        functools.partial(_tc_last, n_nodes=n),
        out_shape=jax.ShapeDtypeStruct((1, Wc2.shape[1]), jnp.float32),
    )(s2, g2, b3, deg_t, Wc1, bc1, Wc2, bc2)
    return logits


# speedup vs baseline: 1.4147x; 1.4147x over previous
import functools

import jax
import jax.numpy as jnp
from jax import lax
from jax.experimental import pallas as pl
from jax.experimental.pallas import tpu as pltpu
from jax.experimental.pallas import tpu_sc as plsc

_NC = 2
_NS = 16
_NW = _NC * _NS
_CH = 112
_SEC = 24
_RING = 3
_TRASH = 112


def _pad_sizes(n_edges, n_nodes):
    nch = -(-n_edges // (_NW * _CH))
    nch = -(-nch // _SEC) * _SEC
    e_pad = _NW * _CH * nch
    n_acc = ((n_nodes + _TRASH + 127) // 128) * 128
    n_deg = ((n_nodes + _TRASH + _NS * 16 - 1) // (_NS * 16)) * (_NS * 16)
    return nch, e_pad, n_acc, n_deg


def _make_deg_kernel(nch, n_acc):
    mesh = plsc.VectorSubcoreMesh(core_axis_name="c", subcore_axis_name="s")
    grp = n_acc // _NS

    @functools.partial(
        pl.kernel,
        out_type=jax.ShapeDtypeStruct((_NC, n_acc), jnp.float32),
        mesh=mesh,
        scratch_types=[
            pltpu.VMEM((nch, _CH), jnp.int32),
            pltpu.VMEM((_CH,), jnp.float32),
            pltpu.VMEM((grp,), jnp.float32),
            pltpu.VMEM_SHARED((n_acc,), jnp.float32),
        ],
    )
    def deg_kernel(dst_hbm, out_hbm, idxd, ones, zbuf, acc):
        cid = lax.axis_index("c")
        sid = lax.axis_index("s")
        wid = cid * _NS + sid
        pltpu.sync_copy(dst_hbm.at[pl.ds(wid * nch, nch)], idxd)

        def fill(i, carry):
            ones[pl.ds(i * 16, 16)] = jnp.ones((16,), jnp.float32)
            return carry
        lax.fori_loop(0, _CH // 16, fill, 0)

        def zfill(i, carry):
            zbuf[pl.ds(i * 16, 16)] = jnp.zeros((16,), jnp.float32)
            return carry
        lax.fori_loop(0, grp // 16, zfill, 0)
        pltpu.sync_copy(zbuf, acc.at[pl.ds(sid * grp, grp)])
        plsc.subcore_barrier()

        def chunk(j, carry):
            pltpu.sync_copy(ones, acc.at[idxd.at[j]], add=True)
            return carry
        lax.fori_loop(0, nch, chunk, 0)
        plsc.subcore_barrier()
        pltpu.sync_copy(acc.at[pl.ds(sid * grp, grp)],
                        out_hbm.at[cid, pl.ds(sid * grp, grp)])

    return deg_kernel


def _make_scatter_kernel(n_nodes, hid, nch, n_acc):
    mesh = plsc.VectorSubcoreMesh(core_axis_name="c", subcore_axis_name="s")
    grp = n_acc // _NS
    assert nch % _SEC == 0 and _SEC % _RING == 0 and _SEC % 8 == 0
    nsec = nch // _SEC
    assert grp % 8 == 0

    @functools.partial(
        pl.kernel,
        out_type=jax.ShapeDtypeStruct((_NC, n_acc, hid), jnp.float32),
        mesh=mesh,
        scratch_types=[
            pltpu.VMEM((_SEC, _CH), jnp.int32),
            pltpu.VMEM((_SEC, _CH), jnp.int32),
        ] + [pltpu.VMEM((_CH, hid), jnp.float32) for _ in range(_RING)]
          + [pltpu.SemaphoreType.DMA for _ in range(_RING)]
          + [pltpu.VMEM_SHARED((n_acc, hid), jnp.float32)],
    )
    def scat_kernel(g_hbm, src_hbm, dst_hbm, out_hbm,
                    idxs, idxd, *bufs_sems_acc):
        rows = bufs_sems_acc[:_RING]
        sems = bufs_sems_acc[_RING:2 * _RING]
        acc = bufs_sems_acc[2 * _RING]
        cid = lax.axis_index("c")
        sid = lax.axis_index("s")
        wid = cid * _NS + sid

        base0 = wid * nch
        pltpu.sync_copy(src_hbm.at[pl.ds(base0, _SEC)], idxs)
        pltpu.sync_copy(dst_hbm.at[pl.ds(base0, _SEC)], idxd)

        def zfill(i, carry):
            for k in range(hid // 16):
                rows[0][i, pl.ds(k * 16, 16)] = jnp.zeros((16,), jnp.float32)
            return carry
        lax.fori_loop(0, _CH, zfill, 0)

        for b in range(1, _RING):
            pltpu.async_copy(g_hbm.at[idxs.at[b]], rows[b], sems[b])
        nfull, rem = divmod(grp, _CH)
        for z in range(nfull):
            pltpu.sync_copy(
                rows[0], acc.at[pl.ds(sid * grp + z * _CH, _CH)])
        if rem:
            pltpu.sync_copy(
                rows[0].at[pl.ds(0, rem)],
                acc.at[pl.ds(sid * grp + nfull * _CH, rem)])
        pltpu.async_copy(g_hbm.at[idxs.at[0]], rows[0], sems[0])
        plsc.subcore_barrier()

        for sec in range(nsec):
            if sec:
                base = wid * nch + sec * _SEC
                pltpu.sync_copy(src_hbm.at[pl.ds(base, _SEC)], idxs)
                pltpu.sync_copy(dst_hbm.at[pl.ds(base, _SEC)], idxd)
                for b in range(_RING):
                    pltpu.async_copy(g_hbm.at[idxs.at[b]], rows[b], sems[b])

            def rnd(p, carry):
                j0 = p * _RING
                for b in range(_RING):
                    j = j0 + b
                    pltpu.make_async_copy(
                        g_hbm.at[idxs.at[j]], rows[b], sems[b]).wait()
                    pltpu.sync_copy(rows[b], acc.at[idxd.at[j]], add=True)

                    @pl.when(j + _RING < _SEC)
                    def _():
                        pltpu.async_copy(
                            g_hbm.at[idxs.at[j + _RING]], rows[b], sems[b])
                return carry
            lax.fori_loop(0, _SEC // _RING, rnd, 0)
        plsc.subcore_barrier()
        pltpu.sync_copy(acc.at[pl.ds(sid * grp, grp)],
                        out_hbm.at[cid, pl.ds(sid * grp, grp)])

    return scat_kernel


def _tc_first(x_ref, wp_ref, bp_ref, w1_ref, degt_ref, g32_ref, *, n_nodes):
    deg = degt_ref[pl.ds(0, n_nodes), 0:1] + degt_ref[pl.ds(0, n_nodes), 1:2] + 1.0
    dinv = lax.rsqrt(deg)
    h0 = jnp.maximum(x_ref[...] @ wp_ref[...] + bp_ref[...][None, :], 0.0)
    g32_ref[...] = dinv * (h0 @ w1_ref[...])


def _tc_mid(s_ref, g_ref, b_ref, w_ref, degt_ref, g32_ref, *, n_nodes):
    deg = degt_ref[pl.ds(0, n_nodes), 0:1] + degt_ref[pl.ds(0, n_nodes), 1:2] + 1.0
    dinv = lax.rsqrt(deg)
    s = s_ref[0, pl.ds(0, n_nodes), :] + s_ref[1, pl.ds(0, n_nodes), :]
    h = jnp.maximum(dinv * (s + g_ref[...]) + b_ref[...][None, :], 0.0)
    g32_ref[...] = dinv * (h @ w_ref[...])


def _tc_last(s_ref, g_ref, b_ref, degt_ref, wc1_ref, bc1_ref, wc2_ref,
             bc2_ref, out_ref, *, n_nodes):
    deg = degt_ref[pl.ds(0, n_nodes), 0:1] + degt_ref[pl.ds(0, n_nodes), 1:2] + 1.0
    dinv = lax.rsqrt(deg)
    s = s_ref[0, pl.ds(0, n_nodes), :] + s_ref[1, pl.ds(0, n_nodes), :]
    h = jnp.maximum(dinv * (s + g_ref[...]) + b_ref[...][None, :], 0.0)
    m = jnp.sum(h, axis=0, keepdims=True) * (1.0 / n_nodes)
    hidden = jnp.maximum(m @ wc1_ref[...] + bc1_ref[...][None, :], 0.0)
    out_ref[...] = hidden @ wc2_ref[...] + bc2_ref[...][None, :]


def kernel(x, edge_index, Wp, bp, W1, b1, W2, b2, W3, b3, Wc1, bc1, Wc2, bc2):
    n = x.shape[0]
    e = edge_index.shape[1]
    hid = W1.shape[0]
    nch, e_pad, n_acc, n_deg = _pad_sizes(e, n)

    src = edge_index[0].astype(jnp.int32)
    dst = edge_index[1].astype(jnp.int32)
    pad = e_pad - e
    src_p = jnp.concatenate(
        [src, (jnp.arange(pad, dtype=jnp.int32) * 37) % n])
    dst_p = jnp.concatenate(
        [dst, n + (jnp.arange(pad, dtype=jnp.int32) % _TRASH)])
    src2d = src_p.reshape(_NW * nch, _CH)
    dst2d = dst_p.reshape(_NW * nch, _CH)

    deg_kernel = _make_deg_kernel(nch, n_deg)
    scat_kernel = _make_scatter_kernel(n, hid, nch, n_acc)

    deg_p = deg_kernel(dst2d)
    deg_t = deg_p.T

    g_shape = jax.ShapeDtypeStruct((n, hid), jnp.float32)
    g0 = pl.pallas_call(
        functools.partial(_tc_first, n_nodes=n),
        out_shape=g_shape,
    )(x, Wp, bp, W1, deg_t)

    s0 = scat_kernel(g0, src2d, dst2d)
    g1 = pl.pallas_call(
        functools.partial(_tc_mid, n_nodes=n),
        out_shape=g_shape,
    )(s0, g0, b1, W2, deg_t)

    s1 = scat_kernel(g1, src2d, dst2d)
    g2 = pl.pallas_call(
        functools.partial(_tc_mid, n_nodes=n),
        out_shape=g_shape,
    )(s1, g1, b2, W3, deg_t)

    s2 = scat_kernel(g2, src2d, dst2d)
    logits = pl.pallas_call(
        functools.partial(_tc_last, n_nodes=n),
        out_shape=jax.ShapeDtypeStruct((1, Wc2.shape[1]), jnp.float32),
    )(s2, g2, b3, deg_t, Wc1, bc1, Wc2, bc2)
    return logits
